# baseline (speedup 1.0000x reference)
import functools
import math

import jax
import jax.numpy as jnp
from jax import lax
from jax.experimental import pallas as pl
from jax.experimental.pallas import tpu as pltpu
from jax.experimental.pallas import tpu_sc as plsc

_NC = 2
_NS = 16
_GC = 128
_BM = 512


def _combine(mean_terms, dir_terms, biases, relu):
    n = (mean_terms + dir_terms)[0][0].shape[0]
    nm, nd = len(mean_terms), len(dir_terms)
    sizes = [len(g) for _, g in dir_terms]
    w = jnp.stack([m[2] for m in mean_terms]
                  + [wi for _, g in dir_terms for wi in g])
    nw = w.shape[0]
    b = jnp.stack(biases)
    b = jnp.concatenate([b, jnp.zeros((8 - b.shape[0], 128), jnp.float32)])

    def body(*refs):
        srefs = refs[:nm]
        crefs = refs[nm:2 * nm]
        xrefs = refs[2 * nm:2 * nm + nd]
        wref, bref, oref = refs[2 * nm + nd], refs[2 * nm + nd + 1], refs[-1]
        acc = jnp.broadcast_to(
            jnp.sum(bref[...], axis=0, keepdims=True), (_BM, 128))
        for i in range(nm):
            inv = 1.0 / jnp.maximum(crefs[i][:, :1], 1.0)
            acc = acc + jnp.dot(srefs[i][...] * inv, wref[i],
                                preferred_element_type=jnp.float32)
        k = nm
        for i in range(nd):
            ws = wref[k]
            for j in range(1, sizes[i]):
                ws = ws + wref[k + j]
            k += sizes[i]
            acc = acc + jnp.dot(xrefs[i][...], ws,
                                preferred_element_type=jnp.float32)
        if relu:
            acc = jnp.maximum(acc, 0.0)
        oref[...] = acc

    in_specs = ([pl.BlockSpec((_BM, 128), lambda i: (i, 0))
                 for _ in range(nm)]
                + [pl.BlockSpec((_BM, 128), lambda i: (i, 0))
                   for _ in range(nm)]
                + [pl.BlockSpec((_BM, 128), lambda i: (i, 0))
                   for _ in range(nd)])
    in_specs.append(pl.BlockSpec((nw, 128, 128), lambda i: (0, 0, 0)))
    in_specs.append(pl.BlockSpec((8, 128), lambda i: (0, 0)))
    args = ([m[0] for m in mean_terms] + [m[1] for m in mean_terms]
            + [d[0] for d in dir_terms] + [w, b])
    return pl.pallas_call(
        body,
        grid=(pl.cdiv(n, _BM),),
        in_specs=in_specs,
        out_specs=pl.BlockSpec((_BM, 128), lambda i: (i, 0)),
        out_shape=jax.ShapeDtypeStruct((n, 128), jnp.float32),
    )(*args)


@functools.lru_cache(maxsize=None)
def _make_segsum(e_pad, nsrc, r, npass, eb, nblk, count_mode=False):
    et = e_pad // _NS
    stripe = r // _NS
    dpad = _NC * npass * r
    mesh = plsc.VectorSubcoreMesh(core_axis_name="c", subcore_axis_name="s")

    @functools.partial(
        pl.kernel,
        out_type=jax.ShapeDtypeStruct((dpad, 128), jnp.float32),
        mesh=mesh,
        compiler_params=pltpu.CompilerParams(needs_layout_passes=False),
        scratch_types=(
            ([] if count_mode else [pltpu.VMEM((eb,), jnp.int32)])
            + [pltpu.VMEM((eb,), jnp.int32)]
            + ([] if count_mode else [pltpu.VMEM((eb + _GC,), jnp.int32)])
            + [pltpu.VMEM((eb + _GC,), jnp.int32)]
            + ([] if count_mode else [pltpu.VMEM((1, _GC), jnp.int32)])
            + [pltpu.VMEM((1, _GC), jnp.int32),
               pltpu.VMEM((_GC, 128), jnp.float32),
               pltpu.VMEM_SHARED((r + 16, 128), jnp.float32),
               pltpu.SemaphoreType.DMA,
               pltpu.SemaphoreType.DMA]
        ),
    )
    def seg(*args):
        if count_mode:
            (dst_hbm, out_hbm, dblk, cdst, dstage, rows, acc,
             sem_g, sem_a) = args
            x_hbm = src_hbm = sblk = csrc = sstage = None
        else:
            (x_hbm, src_hbm, dst_hbm, out_hbm, sblk, dblk, csrc, cdst,
             sstage, dstage, rows, acc, sem_g, sem_a) = args
        c = lax.axis_index("c")
        s = lax.axis_index("s")
        zero16 = jnp.zeros((16,), jnp.float32)
        one16 = jnp.ones((16,), jnp.float32)

        def one_pass(p, _):
            lo = (p * _NC + c) * r
            hi = lo + r
            base = s * stripe

            def fill_rows_zero(i, _):
                for q in range(8):
                    rows[i, pl.ds(q * 16, 16)] = zero16
                return 0
            lax.fori_loop(0, _GC, fill_rows_zero, 0)
            for z in range(stripe // _GC):
                pltpu.sync_copy(rows, acc.at[pl.ds(base + z * _GC, _GC)])

            plsc.subcore_barrier()

            if count_mode:
                def fill_rows_one(i, _):
                    for q in range(8):
                        rows[i, pl.ds(q * 16, 16)] = one16
                    return 0
                lax.fori_loop(0, _GC, fill_rows_one, 0)

            for blk in range(nblk):
                ebase = s * et + blk * eb
                if not count_mode:
                    pltpu.sync_copy(src_hbm.at[pl.ds(ebase, eb)], sblk)
                pltpu.sync_copy(dst_hbm.at[pl.ds(ebase, eb)], dblk)

                def fbody(i, nacc):
                    dv = dblk[pl.ds(i * 16, 16)]
                    m = (dv >= lo) & (dv < hi)
                    mi = m.astype(jnp.int32)
                    off = nacc + jnp.cumsum(mi) - 1
                    if not count_mode:
                        sv = sblk[pl.ds(i * 16, 16)]
                        plsc.store_scatter(csrc, [off], sv, mask=m)
                    plsc.store_scatter(cdst, [off], dv - lo, mask=m)
                    return nacc + jnp.sum(mi)

                nmatch = lax.fori_loop(0, eb // 16, fbody, 0)

                for j in range(_GC // 16):
                    if not count_mode:
                        csrc[pl.ds(nmatch + 16 * j, 16)] = jnp.zeros(
                            (16,), jnp.int32)
                    cdst[pl.ds(nmatch + 16 * j, 16)] = jnp.full(
                        (16,), r, jnp.int32)

                nch = (nmatch + _GC - 1) // _GC

                def gbody(j, _):
                    for q in range(_GC // 16):
                        if not count_mode:
                            sstage[0, pl.ds(q * 16, 16)] = (
                                csrc[pl.ds(j * _GC + q * 16, 16)])
                        dstage[0, pl.ds(q * 16, 16)] = (
                            cdst[pl.ds(j * _GC + q * 16, 16)])
                    if not count_mode:
                        pltpu.async_copy(x_hbm.at[sstage.at[0]], rows,
                                         sem_g).wait()
                    pltpu.async_copy(rows, acc.at[dstage.at[0]],
                                     sem_a, add=True).wait()
                    return 0

                lax.fori_loop(0, nch, gbody, 0)

            plsc.subcore_barrier()

            pltpu.sync_copy(acc.at[pl.ds(base, stripe)],
                            out_hbm.at[pl.ds(lo + base, stripe)])
            return 0

        lax.fori_loop(0, npass, one_pass, 0)

    return seg


def _edge_blocks(e):
    nblk = max(1, math.ceil(e / (_NS * 3920)))
    eb = math.ceil(e / (_NS * nblk * 16)) * 16
    return eb, nblk


def _pad_edges(idx, e_pad, fill):
    e = idx.shape[0]
    if e_pad > e:
        idx = jnp.concatenate(
            [idx, jnp.full((e_pad - e,), fill, jnp.int32)])
    return idx


def _plan(num_dst, budget_words, row_words):
    rmax = ((budget_words // row_words - 16) // 2048) * 2048
    npass = 1
    while math.ceil(num_dst / (_NC * npass) / 2048) * 2048 > rmax:
        npass += 1
    r = math.ceil(num_dst / (_NC * npass) / 2048) * 2048
    return r, npass


def _segsum(x, src, dst, num_dst):
    eb, nblk = _edge_blocks(src.shape[0])
    e_pad = _NS * nblk * eb
    src = _pad_edges(src, e_pad, 0)
    dst = _pad_edges(dst, e_pad, -1)
    vmem_words = 4 * eb + 2 * _GC + 512 + 128 * _GC + 30656
    r, npass = _plan(num_dst, 2097151 - _NS * vmem_words - 8192, 128)
    out = _make_segsum(e_pad, x.shape[0], r, npass, eb, nblk)(x, src, dst)
    return out[:num_dst]


def _segcount(dst, num_dst):
    eb, nblk = _edge_blocks(dst.shape[0])
    e_pad = _NS * nblk * eb
    dst = _pad_edges(dst, e_pad, -1)
    vmem_words = 4 * eb + 2 * _GC + 512 + 128 * _GC + 30656
    r, npass = _plan(num_dst, 2097151 - _NS * vmem_words - 8192, 128)
    out = _make_segsum(e_pad, 0, r, npass, eb, nblk, count_mode=True)(dst)
    return out[:num_dst]


def kernel(x_news, x_keyword, x_stock, ei_nk_src, ei_nk_dst, ei_ns_src,
           ei_ns_dst, W_news, b_news, W_kw, b_kw, W_st, b_st,
           c1_Wl, c1_bl, c1_Wr, c2_Wl, c2_bl, c2_Wr):
    nn, nk, ns = x_news.shape[0], x_keyword.shape[0], x_stock.shape[0]

    xn = _combine([], [(x_news, [W_news])], [b_news], True)
    xk = _combine([], [(x_keyword, [W_kw])], [b_kw], True)
    xs = _combine([], [(x_stock, [W_st])], [b_st], True)

    cnt_nk_d = _segcount(ei_nk_dst, nk)
    cnt_nk_s = _segcount(ei_nk_src, nn)
    cnt_ns_d = _segcount(ei_ns_dst, ns)
    cnt_ns_s = _segcount(ei_ns_src, nn)

    def hetero(a_n, a_k, a_s, Wl, bl, Wr, relu):
        sk = _segsum(a_n, ei_nk_src, ei_nk_dst, nk)
        sn1 = _segsum(a_k, ei_nk_dst, ei_nk_src, nn)
        sn3 = _segsum(a_s, ei_ns_dst, ei_ns_src, nn)
        ss = _segsum(a_n, ei_ns_src, ei_ns_dst, ns)
        out_k = _combine([(sk, cnt_nk_d, Wl[0])], [(a_k, [Wr[0]])],
                         [bl[0]], relu)
        out_n = _combine([(sn1, cnt_nk_s, Wl[1]), (sn3, cnt_ns_s, Wl[3])],
                         [(a_n, [Wr[1], Wr[3]])], [bl[1], bl[3]], relu)
        out_s = _combine([(ss, cnt_ns_d, Wl[2])], [(a_s, [Wr[2]])],
                         [bl[2]], relu)
        return out_n, out_k, out_s

    n1, k1, s1 = hetero(xn, xk, xs, c1_Wl, c1_bl, c1_Wr, True)
    n2, k2, s2 = hetero(n1, k1, s1, c2_Wl, c2_bl, c2_Wr, False)
    return n2, k2, s2
